# baseline (speedup 1.0000x reference)
import jax
import jax.numpy as jnp
from jax.experimental import pallas as pl
from jax.experimental.pallas import tpu as pltpu

KK = 25
PAD = 12
WOFF = 16
RT = 16
CT = 8
NB = 256
SH = 40
SW = 40
KDIM = SH * SW


def _round_up(x, m):
    return ((x + m - 1) // m) * m


def _banded_a(weight):
    w2 = weight.reshape(KK, KK).astype(jnp.float32)
    dxs = jnp.arange(KK)
    cs = jnp.arange(CT)
    wls = jnp.arange(SW)
    xm = (wls[None, None, :] == cs[None, :, None] + dxs[:, None, None] + 4)
    dys = jnp.arange(KK)
    rs = jnp.arange(RT)
    hls = jnp.arange(SH)
    ym = (hls[None, None, :] == rs[None, :, None] + dys[:, None, None])
    t1 = jnp.einsum('yx,xcw->ycw', w2, xm.astype(jnp.float32))
    a4 = jnp.einsum('ycw,yrh->rchw', t1, ym.astype(jnp.float32))
    return a4.reshape(RT * CT, KDIM).astype(jnp.bfloat16)


def _make_kernel(H, W):

    def _conv_sig_kernel(a_ref, b_ref, x_ref, o_ref, xp_ref, band_ref):
        h_id = pl.program_id(1)
        Hp, Wp, _ = xp_ref.shape

        @pl.when(h_id == 0)
        def _build():
            xp_ref[:PAD, :, :] = jnp.zeros((PAD, Wp, NB), jnp.bfloat16)
            xp_ref[PAD + H:, :, :] = jnp.zeros(
                (Hp - PAD - H, Wp, NB), jnp.bfloat16)
            xp_ref[PAD:PAD + H, :WOFF, :] = jnp.zeros((H, WOFF, NB),
                                                      jnp.bfloat16)
            xp_ref[PAD:PAD + H, WOFF + W:, :] = jnp.zeros(
                (H, Wp - WOFF - W, NB), jnp.bfloat16)
            for h0 in range(0, H, 8):
                for ns in range(0, NB, 128):
                    piece = x_ref[ns:ns + 128, h0 * W:(h0 + 8) * W]
                    t = piece.T.reshape(8, W, 128).astype(jnp.bfloat16)
                    xp_ref[PAD + h0:PAD + h0 + 8, WOFF:WOFF + W,
                           ns:ns + 128] = t

        bias = b_ref[0]
        a = a_ref[...]
        hb = h_id * RT
        for wt in range(W // CT):
            wb = wt * CT
            slab = xp_ref[pl.ds(hb, SH), wb:wb + SW, :].reshape(KDIM, NB)
            acc = jnp.dot(a, slab, preferred_element_type=jnp.float32)
            band_ref[:, wb:wb + CT, :] = (
                jax.nn.sigmoid(acc + bias).reshape(RT, CT, NB))

        for j in range(RT):
            for ns in range(0, NB, 128):
                piece = band_ref[j, :, ns:ns + 128]
                o_ref[pl.ds(ns, 128), j * W:(j + 1) * W] = piece.T

    return _conv_sig_kernel


def _forward(x_nchw, weight, bias):
    N, C, H, W = x_nchw.shape
    assert C == 1
    Hp = _round_up(PAD + H + PAD, 8)
    Wp = _round_up(WOFF + W + PAD, 8)

    a_mat = _banded_a(weight)

    x2 = x_nchw.reshape(N, H * W)
    Np = _round_up(N, NB)
    if Np != N:
        x2 = jnp.pad(x2, ((0, Np - N), (0, 0)))

    out = pl.pallas_call(
        _make_kernel(H, W),
        out_shape=jax.ShapeDtypeStruct((Np, H * W), x_nchw.dtype),
        grid=(Np // NB, H // RT),
        in_specs=[
            pl.BlockSpec((RT * CT, KDIM), lambda b, h: (0, 0)),
            pl.BlockSpec(memory_space=pltpu.MemorySpace.SMEM),
            pl.BlockSpec((NB, H * W), lambda b, h: (b, 0)),
        ],
        out_specs=pl.BlockSpec((NB, RT * W), lambda b, h: (b, h)),
        scratch_shapes=[
            pltpu.VMEM((Hp, Wp, NB), jnp.bfloat16),
            pltpu.VMEM((RT, W, NB), jnp.float32),
        ],
        compiler_params=pltpu.CompilerParams(
            dimension_semantics=("parallel", "arbitrary")),
    )(a_mat, bias.astype(jnp.float32), x2)

    return out[:N].reshape(N, 1, H, W)


def kernel(x_nchw, weight, bias):
    return _forward(x_nchw, weight, bias)
